# baseline (speedup 1.0000x reference)
import jax
import jax.numpy as jnp
from jax import lax
from jax.experimental import pallas as pl
from jax.experimental.pallas import tpu as pltpu
from jax.experimental.pallas import tpu_sc as plsc

_BATCH = 100
_DIM = 1024
_MARGIN = 0.1
_NS = 16
_RPW = 8
_PAD = _NS * _RPW
_LANES = 16
_CHUNKS = _DIM // _LANES


def _lane_permute(x, idx):
    dnums = lax.GatherDimensionNumbers(
        offset_dims=(), collapsed_slice_dims=(0,), start_index_map=(0,)
    )
    return lax.gather(
        x, idx[:, None], dnums, slice_sizes=(1,),
        mode=lax.GatherScatterMode.PROMISE_IN_BOUNDS,
    )


_mesh = plsc.VectorSubcoreMesh(
    core_axis_name="c", subcore_axis_name="s", num_cores=1, num_subcores=_NS
)

_SCRATCH = [
    pltpu.VMEM((_RPW,), jnp.int32),
    pltpu.VMEM((_RPW,), jnp.int32),
    pltpu.VMEM((_RPW, _DIM), jnp.float32),
    pltpu.VMEM((_RPW, _DIM), jnp.float32),
    pltpu.VMEM((_RPW, _DIM), jnp.float32),
    pltpu.VMEM((_LANES,), jnp.float32),
    pltpu.SemaphoreType.DMA,
    pltpu.SemaphoreType.DMA,
]


def _loss_body(outputs_hbm, labels_hbm, rand_hbm, emb_hbm, out_hbm,
               idx_t, idx_n, outs, rows_t, rows_n, part_v, sem_t, sem_n):
    sid = lax.axis_index("s")
    base = sid * _RPW
    pltpu.sync_copy(labels_hbm.at[pl.ds(base, _RPW)], idx_t)
    pltpu.sync_copy(rand_hbm.at[pl.ds(base, _RPW)], idx_n)
    cp_t = pltpu.async_copy(emb_hbm.at[idx_t], rows_t, sem_t)
    cp_n = pltpu.async_copy(emb_hbm.at[idx_n], rows_n, sem_n)
    pltpu.sync_copy(outputs_hbm.at[pl.ds(base, _RPW)], outs)
    cp_t.wait()
    cp_n.wait()

    rot_idx = [
        (lax.iota(jnp.int32, _LANES) + s) & (_LANES - 1) for s in (8, 4, 2, 1)
    ]
    partial_vec = jnp.zeros((_LANES,), jnp.float32)
    for r in range(_RPW):
        diff = jnp.zeros((_LANES,), jnp.float32)
        for j in range(_CHUNKS):
            o = outs[r, pl.ds(j * _LANES, _LANES)]
            t = rows_t[r, pl.ds(j * _LANES, _LANES)]
            n = rows_n[r, pl.ds(j * _LANES, _LANES)]
            diff = diff + o * (n - t)
        for idx in rot_idx:
            diff = diff + _lane_permute(diff, idx)
        row_loss = jnp.maximum(jnp.float32(_MARGIN) + diff, 0.0)
        gate = jnp.where((base + r) < _BATCH, jnp.float32(1.0), jnp.float32(0.0))
        partial_vec = partial_vec + row_loss * jnp.broadcast_to(gate, (_LANES,))

    part_v[...] = partial_vec
    pltpu.sync_copy(part_v, out_hbm.at[sid])


_loss_kernel = pl.kernel(
    _loss_body,
    out_type=jax.ShapeDtypeStruct((_NS, _LANES), jnp.float32),
    mesh=_mesh,
    scratch_types=_SCRATCH,
)


def _sum_body(parts_ref, out_ref):
    out_ref[0, 0] = jnp.sum(parts_ref[:, 0:1])


_sum_kernel = pl.pallas_call(
    _sum_body,
    out_shape=jax.ShapeDtypeStruct((1, 1), jnp.float32),
    in_specs=[pl.BlockSpec(memory_space=pltpu.VMEM)],
    out_specs=pl.BlockSpec(memory_space=pltpu.SMEM),
)


def kernel(outputs, labels, labels_random, embeddings):
    pad = _PAD - _BATCH
    outputs_p = jnp.pad(outputs, ((0, pad), (0, 0)))
    labels_p = jnp.pad(labels, (0, pad))
    rand_p = jnp.pad(labels_random, (0, pad))
    parts = _loss_kernel(outputs_p, labels_p, rand_p, embeddings)
    return _sum_kernel(parts)[0, 0]


# speedup vs baseline: 1.2826x; 1.2826x over previous
import jax
import jax.numpy as jnp
from jax import lax
from jax.experimental import pallas as pl
from jax.experimental.pallas import tpu as pltpu
from jax.experimental.pallas import tpu_sc as plsc

_BATCH = 100
_DIM = 1024
_MARGIN = 0.1
_NS = 16
_RPW = 8
_PAD = _NS * _RPW
_LANES = 16
_CHUNKS = _DIM // _LANES

_mesh = plsc.VectorSubcoreMesh(
    core_axis_name="c", subcore_axis_name="s", num_cores=1, num_subcores=_NS
)

_SCRATCH = [
    pltpu.VMEM((_RPW,), jnp.int32),
    pltpu.VMEM((_RPW,), jnp.int32),
    pltpu.VMEM((_RPW, _DIM), jnp.float32),
    pltpu.VMEM((_RPW, _DIM), jnp.float32),
    pltpu.VMEM((_RPW, _DIM), jnp.float32),
    pltpu.VMEM((_RPW, _LANES), jnp.float32),
    pltpu.SemaphoreType.DMA,
    pltpu.SemaphoreType.DMA,
]


def _loss_body(outputs_hbm, labels_hbm, rand_hbm, emb_hbm, out_hbm,
               idx_t, idx_n, outs, rows_t, rows_n, drows, sem_t, sem_n):
    sid = lax.axis_index("s")
    base = sid * _RPW
    pltpu.sync_copy(labels_hbm.at[pl.ds(base, _RPW)], idx_t)
    pltpu.sync_copy(rand_hbm.at[pl.ds(base, _RPW)], idx_n)
    cp_t = pltpu.async_copy(emb_hbm.at[idx_t], rows_t, sem_t)
    cp_n = pltpu.async_copy(emb_hbm.at[idx_n], rows_n, sem_n)
    pltpu.sync_copy(outputs_hbm.at[pl.ds(base, _RPW)], outs)
    cp_t.wait()
    cp_n.wait()

    zero = jnp.zeros((_LANES,), jnp.float32)

    def body(j, accs):
        col = j * _LANES
        new = []
        for r in range(_RPW):
            o = outs[r, pl.ds(col, _LANES)]
            t = rows_t[r, pl.ds(col, _LANES)]
            n = rows_n[r, pl.ds(col, _LANES)]
            new.append(accs[r] + o * (n - t))
        return tuple(new)

    accs = lax.fori_loop(0, _CHUNKS, body, (zero,) * _RPW)
    for r in range(_RPW):
        drows[r] = accs[r]
    pltpu.sync_copy(drows, out_hbm.at[sid])


_loss_kernel = pl.kernel(
    _loss_body,
    out_type=jax.ShapeDtypeStruct((_NS, _RPW, _LANES), jnp.float32),
    mesh=_mesh,
    scratch_types=_SCRATCH,
)


def _sum_body(parts_ref, out_ref):
    d = jnp.sum(parts_ref[...], axis=1)
    loss = jnp.maximum(jnp.float32(_MARGIN) + d, 0.0)
    row = lax.broadcasted_iota(jnp.int32, (_PAD,), 0)
    loss = jnp.where(row < _BATCH, loss, 0.0)
    out_ref[0, 0] = jnp.sum(loss)


_sum_kernel = pl.pallas_call(
    _sum_body,
    out_shape=jax.ShapeDtypeStruct((1, 1), jnp.float32),
    in_specs=[pl.BlockSpec(memory_space=pltpu.VMEM)],
    out_specs=pl.BlockSpec(memory_space=pltpu.SMEM),
)


def kernel(outputs, labels, labels_random, embeddings):
    pad = _PAD - _BATCH
    outputs_p = jnp.pad(outputs, ((0, pad), (0, 0)))
    labels_p = jnp.pad(labels, (0, pad))
    rand_p = jnp.pad(labels_random, (0, pad))
    parts = _loss_kernel(outputs_p, labels_p, rand_p, embeddings)
    return _sum_kernel(parts.reshape(_PAD, _LANES))[0, 0]
